# baseline (speedup 1.0000x reference)
import numpy as np
import jax
import jax.numpy as jnp
from jax import lax
from jax.experimental import pallas as pl
from jax.experimental.pallas import tpu as pltpu
from jax.experimental.pallas import tpu_sc as plsc

FEAT_STRIDE = 16
PRE_NMS_TOPN = 6000
POST_NMS_TOPN = 300
NMS_THRESH = 0.7
A_NUM = 9
B, H, W = 4, 50, 50
N = H * W * A_NUM
NP = 32768
C = 128
R = NP // C
TOPP = 6016
GCH = TOPP // C
KOUT = 512
L = 16
NC, NS = 2, 16


def _gen_all_anchors():
    base_size = 16.0
    ratios = np.array([0.5, 1.0, 2.0], dtype=np.float64)
    scales = np.array([8.0, 16.0, 32.0], dtype=np.float64)
    base = np.array([1, 1, base_size, base_size], dtype=np.float64) - 1
    w = base[2] - base[0] + 1
    h = base[3] - base[1] + 1
    x_ctr = base[0] + 0.5 * (w - 1)
    y_ctr = base[1] + 0.5 * (h - 1)
    size = w * h
    ws_r = np.round(np.sqrt(size / ratios))
    hs_r = np.round(ws_r * ratios)
    rows = []
    for i in range(3):
        ws = ws_r[i] * scales
        hs = hs_r[i] * scales
        for j in range(3):
            rows.append([x_ctr - 0.5 * (ws[j] - 1), y_ctr - 0.5 * (hs[j] - 1),
                         x_ctr + 0.5 * (ws[j] - 1), y_ctr + 0.5 * (hs[j] - 1)])
    anc = np.array(rows, dtype=np.float32)
    ys, xs = np.meshgrid(np.arange(H), np.arange(W), indexing="ij")
    sx = (xs * FEAT_STRIDE).ravel().astype(np.float32)
    sy = (ys * FEAT_STRIDE).ravel().astype(np.float32)
    shifts = np.stack([sx, sy, sx, sy], axis=1)
    full = (anc[None, :, :] + shifts[:, None, :]).reshape(N, 4)
    out = np.zeros((NP, 4), dtype=np.float32)
    out[:N] = full
    return out


_ANCHORS = _gen_all_anchors()


def _xor_rows(x, g):
    xr = x.reshape(B, R // (2 * g), 2, g, C)
    sw = jnp.concatenate([xr[:, :, 1:2], xr[:, :, 0:1]], axis=2)
    return sw.reshape(B, R, C)


def _xor_lanes(x, g, ci):
    up = pltpu.roll(x, C - g, 2)
    dn = pltpu.roll(x, g, 2)
    return jnp.where((ci & g) == 0, up, dn)


def _tc_body(iminfo_ref, sc_ref, dx_ref, dy_ref, dw_ref, dh_ref,
             ax1_ref, ay1_ref, ax2_ref, ay2_ref,
             key_ref, idx_ref, bx1_ref, by1_ref, bx2_ref, by2_ref):
    ax1 = ax1_ref[...]
    ay1 = ay1_ref[...]
    ax2 = ax2_ref[...]
    ay2 = ay2_ref[...]
    widths = ax2 - ax1 + 1.0
    heights = ay2 - ay1 + 1.0
    ctr_x = ax1 + 0.5 * widths
    ctr_y = ay1 + 0.5 * heights
    for b in range(B):
        pcx = dx_ref[b] * widths + ctr_x
        pcy = dy_ref[b] * heights + ctr_y
        pw = jnp.exp(dw_ref[b]) * widths
        ph = jnp.exp(dh_ref[b]) * heights
        x1 = pcx - 0.5 * pw
        y1 = pcy - 0.5 * ph
        x2 = pcx + 0.5 * pw
        y2 = pcy + 0.5 * ph
        hmax = iminfo_ref[b, 0] - 1.0
        wmax = iminfo_ref[b, 1] - 1.0
        bx1_ref[b] = jnp.clip(x1, 0.0, wmax)
        by1_ref[b] = jnp.clip(y1, 0.0, hmax)
        bx2_ref[b] = jnp.clip(x2, 0.0, wmax)
        by2_ref[b] = jnp.clip(y2, 0.0, hmax)

    ri = lax.broadcasted_iota(jnp.int32, (B, R, C), 1)
    ci = lax.broadcasted_iota(jnp.int32, (B, R, C), 2)
    bi = lax.broadcasted_iota(jnp.int32, (B, R, C), 0)
    pos = ri * C + ci
    key_ref[...] = sc_ref[...]
    idx_ref[...] = pos + bi * NP
    one = jnp.int32(1)

    def sort_pass(jv, kv, gv, axis):
        io = ri if axis == 1 else ci
        size = R if axis == 1 else C
        key = key_ref[...]
        idx = idx_ref[...]
        sel = (io & gv) == 0
        pk = jnp.where(sel, pltpu.roll(key, size - gv, axis),
                       pltpu.roll(key, gv, axis))
        pi = jnp.where(sel, pltpu.roll(idx, size - gv, axis),
                       pltpu.roll(idx, gv, axis))
        lower = (pos & jv) == 0
        region = (pos & kv) == 0
        self_first = (key > pk) | ((key == pk) & (idx < pi))
        take_self = (self_first == lower) == region
        key_ref[...] = jnp.where(take_self, key, pk)
        idx_ref[...] = jnp.where(take_self, idx, pi)

    def stage(s, carry):
        kv = lax.shift_left(one, s)

        def row_pass(i, c2):
            tt = s - 1 - i
            sort_pass(lax.shift_left(one, tt), kv,
                      lax.shift_left(one, tt - 7), axis=1)
            return c2

        lax.fori_loop(0, jnp.maximum(s - 7, 0), row_pass, 0)

        def lane_pass(i, c2):
            jv = lax.shift_left(one, jnp.minimum(s - 1, 6) - i)
            sort_pass(jv, kv, jv, axis=2)
            return c2

        lax.fori_loop(0, jnp.minimum(s, 7), lane_pass, 0)
        return carry

    lax.fori_loop(1, 16, stage, 0)


def _sc_body(sidx_hbm, ssc_hbm, box_hbm,
             okx1, oky1, okx2, oky2, oks,
             idx_v, sc_v, gbox_v,
             kx1_v, ky1_v, kx2_v, ky2_v, kar_v, ks_v,
             sem):
    cid = lax.axis_index("c")
    sid = lax.axis_index("s")
    wid = cid * NS + sid

    @pl.when(wid < B)
    def _():
        b = wid
        pltpu.sync_copy(sidx_hbm.at[b], idx_v)
        pltpu.sync_copy(ssc_hbm.at[b], sc_v.at[pl.ds(0, TOPP)])
        cps = []
        for g in range(GCH):
            cps.append(pltpu.async_copy(
                box_hbm.at[idx_v.at[g]], gbox_v.at[pl.ds(g * C, C)], sem))
        for cp in cps:
            cp.wait()
        zf = jnp.zeros((L,), jnp.float32)
        for i in range(KOUT // L):
            sl = pl.ds(i * L, L)
            kx1_v[sl] = zf
            ky1_v[sl] = zf
            kx2_v[sl] = zf
            ky2_v[sl] = zf
            kar_v[sl] = zf
            ks_v[sl] = zf
    lane = lax.iota(jnp.int32, L)
    lane0 = lane == 0
    t_max = jnp.where(wid < B, PRE_NMS_TOPN, 0)

    def nms_cond(st):
        t, n, jj = st
        return jnp.logical_and(t < t_max, n < POST_NMS_TOPN)

    def nms_body(st):
        t, n, jj = st
        row = gbox_v[t]
        vx1 = jnp.full((L,), row[0])
        vy1 = jnp.full((L,), row[1])
        vx2 = jnp.full((L,), row[2])
        vy2 = jnp.full((L,), row[3])
        varc = (vx2 - vx1 + 1.0) * (vy2 - vy1 + 1.0)
        sl = pl.ds(jj * L, L)
        xx1 = jnp.maximum(kx1_v[sl], vx1)
        yy1 = jnp.maximum(ky1_v[sl], vy1)
        xx2 = jnp.minimum(kx2_v[sl], vx2)
        yy2 = jnp.minimum(ky2_v[sl], vy2)
        ww = jnp.maximum(0.0, xx2 - xx1 + 1.0)
        hh = jnp.maximum(0.0, yy2 - yy1 + 1.0)
        inter = ww * hh
        iou = inter / ((kar_v[sl] + varc) - inter)
        m = (iou > NMS_THRESH) & ((jj * L + lane) < n)
        sup = jnp.any(m)
        done = jnp.logical_or(sup, (jj + 1) * L >= n)
        keep = jnp.logical_and(done, jnp.logical_not(sup))
        kmask = lane0 & keep
        nvec = jnp.full((L,), n, dtype=jnp.int32)
        plsc.store_scatter(kx1_v, [nvec], vx1, mask=kmask)
        plsc.store_scatter(ky1_v, [nvec], vy1, mask=kmask)
        plsc.store_scatter(kx2_v, [nvec], vx2, mask=kmask)
        plsc.store_scatter(ky2_v, [nvec], vy2, mask=kmask)
        plsc.store_scatter(kar_v, [nvec], varc, mask=kmask)
        plsc.store_scatter(ks_v, [nvec], sc_v[pl.ds(t, L)], mask=kmask)
        t2 = jnp.where(done, t + 1, t)
        n2 = n + keep.astype(jnp.int32)
        jj2 = jnp.where(done, 0, jj + 1)
        return t2, n2, jj2

    lax.while_loop(nms_cond, nms_body,
                   (jnp.int32(0), jnp.int32(0), jnp.int32(0)))

    @pl.when(wid < B)
    def _():
        b = wid
        pltpu.sync_copy(kx1_v, okx1.at[b])
        pltpu.sync_copy(ky1_v, oky1.at[b])
        pltpu.sync_copy(kx2_v, okx2.at[b])
        pltpu.sync_copy(ky2_v, oky2.at[b])
        pltpu.sync_copy(ks_v, oks.at[b])


def _prep(scores, bbox_deltas):
    sc = scores[:, A_NUM:, :, :].transpose(0, 2, 3, 1).reshape(B, N)
    dl = bbox_deltas.transpose(0, 2, 3, 1).reshape(B, N, 4)
    sc_p = jnp.concatenate(
        [sc, jnp.full((B, NP - N), -jnp.inf, jnp.float32)], axis=1)
    dl_p = jnp.concatenate(
        [dl, jnp.zeros((B, NP - N, 4), jnp.float32)], axis=1)
    arrs = {
        "sc": sc_p.reshape(B, R, C),
        "dx": dl_p[..., 0].reshape(B, R, C),
        "dy": dl_p[..., 1].reshape(B, R, C),
        "dw": dl_p[..., 2].reshape(B, R, C),
        "dh": dl_p[..., 3].reshape(B, R, C),
        "ax1": jnp.asarray(_ANCHORS[:, 0].reshape(R, C)),
        "ay1": jnp.asarray(_ANCHORS[:, 1].reshape(R, C)),
        "ax2": jnp.asarray(_ANCHORS[:, 2].reshape(R, C)),
        "ay2": jnp.asarray(_ANCHORS[:, 3].reshape(R, C)),
    }
    return arrs


def _tc_call(im_info, a, interpret=False):
    f32 = jnp.float32
    out_shape = [
        jax.ShapeDtypeStruct((B, R, C), f32),
        jax.ShapeDtypeStruct((B, R, C), jnp.int32),
        jax.ShapeDtypeStruct((B, R, C), f32),
        jax.ShapeDtypeStruct((B, R, C), f32),
        jax.ShapeDtypeStruct((B, R, C), f32),
        jax.ShapeDtypeStruct((B, R, C), f32),
    ]
    in_specs = [pl.BlockSpec(memory_space=pltpu.SMEM)] + \
        [pl.BlockSpec(memory_space=pltpu.VMEM)] * 9
    fn = pl.pallas_call(_tc_body, out_shape=out_shape, in_specs=in_specs,
                        interpret=interpret)
    return fn(im_info, a["sc"], a["dx"], a["dy"], a["dw"], a["dh"],
              a["ax1"], a["ay1"], a["ax2"], a["ay2"])


def kernel(scores, bbox_deltas, im_info):
    a = _prep(scores, bbox_deltas)
    key, idx, bx1, by1, bx2, by2 = _tc_call(im_info, a)

    sidx = idx.reshape(B, NP)[:, :TOPP].reshape(B, GCH, C)
    ssc = key.reshape(B, NP)[:, :TOPP]
    box4 = jnp.stack(
        [bx1.reshape(-1), by1.reshape(-1), bx2.reshape(-1), by2.reshape(-1)],
        axis=1)
    box16 = jnp.tile(box4, (1, 4))

    f32 = jnp.float32
    row = jax.ShapeDtypeStruct((B, KOUT), f32)
    nms = pl.kernel(
        _sc_body,
        out_type=[row, row, row, row, row],
        mesh=plsc.VectorSubcoreMesh(core_axis_name="c", subcore_axis_name="s"),
        compiler_params=pltpu.CompilerParams(needs_layout_passes=False,
                                             use_tc_tiling_on_sc=False),
        scratch_types=[pltpu.VMEM((GCH, C), jnp.int32),
                       pltpu.VMEM((TOPP + L,), f32),
                       pltpu.VMEM((TOPP, L), f32)]
        + [pltpu.VMEM((KOUT,), f32)] * 6
        + [pltpu.SemaphoreType.DMA],
    )
    kx1o, ky1o, kx2o, ky2o, kso = nms(sidx, ssc, box16)

    kept = jnp.stack([kx1o, ky1o, kx2o, ky2o], axis=-1)[:, :POST_NMS_TOPN]
    bcol = jnp.broadcast_to(
        jnp.arange(B, dtype=f32)[:, None, None], (B, POST_NMS_TOPN, 1))
    output = jnp.concatenate([bcol, kept], axis=2)
    scores_single = kso[B - 1, :POST_NMS_TOPN].reshape(-1, 1)
    return output, scores_single


# speedup vs baseline: 1.0826x; 1.0826x over previous
import numpy as np
import jax
import jax.numpy as jnp
from jax import lax
from jax.experimental import pallas as pl
from jax.experimental.pallas import tpu as pltpu
from jax.experimental.pallas import tpu_sc as plsc

FEAT_STRIDE = 16
PRE_NMS_TOPN = 6000
POST_NMS_TOPN = 300
NMS_THRESH = 0.7
A_NUM = 9
B, H, W = 4, 50, 50
N = H * W * A_NUM
NP = 32768
C = 128
R = NP // C
TOPP = 6016
GCH = TOPP // C
KOUT = 512
L = 16
NC, NS = 2, 16


def _gen_all_anchors():
    base_size = 16.0
    ratios = np.array([0.5, 1.0, 2.0], dtype=np.float64)
    scales = np.array([8.0, 16.0, 32.0], dtype=np.float64)
    base = np.array([1, 1, base_size, base_size], dtype=np.float64) - 1
    w = base[2] - base[0] + 1
    h = base[3] - base[1] + 1
    x_ctr = base[0] + 0.5 * (w - 1)
    y_ctr = base[1] + 0.5 * (h - 1)
    size = w * h
    ws_r = np.round(np.sqrt(size / ratios))
    hs_r = np.round(ws_r * ratios)
    rows = []
    for i in range(3):
        ws = ws_r[i] * scales
        hs = hs_r[i] * scales
        for j in range(3):
            rows.append([x_ctr - 0.5 * (ws[j] - 1), y_ctr - 0.5 * (hs[j] - 1),
                         x_ctr + 0.5 * (ws[j] - 1), y_ctr + 0.5 * (hs[j] - 1)])
    anc = np.array(rows, dtype=np.float32)
    ys, xs = np.meshgrid(np.arange(H), np.arange(W), indexing="ij")
    sx = (xs * FEAT_STRIDE).ravel().astype(np.float32)
    sy = (ys * FEAT_STRIDE).ravel().astype(np.float32)
    shifts = np.stack([sx, sy, sx, sy], axis=1)
    full = (anc[None, :, :] + shifts[:, None, :]).reshape(N, 4)
    out = np.zeros((NP, 4), dtype=np.float32)
    out[:N] = full
    return out


_ANCHORS = _gen_all_anchors()


def _xor_rows(x, g):
    xr = x.reshape(B, R // (2 * g), 2, g, C)
    sw = jnp.concatenate([xr[:, :, 1:2], xr[:, :, 0:1]], axis=2)
    return sw.reshape(B, R, C)


def _xor_lanes(x, g, ci):
    up = pltpu.roll(x, C - g, 2)
    dn = pltpu.roll(x, g, 2)
    return jnp.where((ci & g) == 0, up, dn)


def _tc_body(iminfo_ref, sc_ref, dx_ref, dy_ref, dw_ref, dh_ref,
             ax1_ref, ay1_ref, ax2_ref, ay2_ref,
             key_ref, idx_ref, bx1_ref, by1_ref, bx2_ref, by2_ref):
    ax1 = ax1_ref[...]
    ay1 = ay1_ref[...]
    ax2 = ax2_ref[...]
    ay2 = ay2_ref[...]
    widths = ax2 - ax1 + 1.0
    heights = ay2 - ay1 + 1.0
    ctr_x = ax1 + 0.5 * widths
    ctr_y = ay1 + 0.5 * heights
    for b in range(B):
        pcx = dx_ref[b] * widths + ctr_x
        pcy = dy_ref[b] * heights + ctr_y
        pw = jnp.exp(dw_ref[b]) * widths
        ph = jnp.exp(dh_ref[b]) * heights
        x1 = pcx - 0.5 * pw
        y1 = pcy - 0.5 * ph
        x2 = pcx + 0.5 * pw
        y2 = pcy + 0.5 * ph
        hmax = iminfo_ref[b, 0] - 1.0
        wmax = iminfo_ref[b, 1] - 1.0
        bx1_ref[b] = jnp.clip(x1, 0.0, wmax)
        by1_ref[b] = jnp.clip(y1, 0.0, hmax)
        bx2_ref[b] = jnp.clip(x2, 0.0, wmax)
        by2_ref[b] = jnp.clip(y2, 0.0, hmax)

    ri = lax.broadcasted_iota(jnp.int32, (B, R, C), 1)
    ci = lax.broadcasted_iota(jnp.int32, (B, R, C), 2)
    bi = lax.broadcasted_iota(jnp.int32, (B, R, C), 0)
    pos = ri * C + ci
    key_ref[...] = sc_ref[...]
    idx_ref[...] = pos + bi * NP
    one = jnp.int32(1)

    def sort_pass(jv, kv, gv, axis):
        io = ri if axis == 1 else ci
        size = R if axis == 1 else C
        key = key_ref[...]
        idx = idx_ref[...]
        sel = (io & gv) == 0
        pk = jnp.where(sel, pltpu.roll(key, size - gv, axis),
                       pltpu.roll(key, gv, axis))
        pi = jnp.where(sel, pltpu.roll(idx, size - gv, axis),
                       pltpu.roll(idx, gv, axis))
        lower = (pos & jv) == 0
        region = (pos & kv) == 0
        self_first = (key > pk) | ((key == pk) & (idx < pi))
        take_self = (self_first == lower) == region
        key_ref[...] = jnp.where(take_self, key, pk)
        idx_ref[...] = jnp.where(take_self, idx, pi)

    def stage(s, carry):
        kv = lax.shift_left(one, s)

        def row_pass(i, c2):
            tt = s - 1 - i
            sort_pass(lax.shift_left(one, tt), kv,
                      lax.shift_left(one, tt - 7), axis=1)
            return c2

        lax.fori_loop(0, jnp.maximum(s - 7, 0), row_pass, 0)

        def lane_pass(i, c2):
            jv = lax.shift_left(one, jnp.minimum(s - 1, 6) - i)
            sort_pass(jv, kv, jv, axis=2)
            return c2

        lax.fori_loop(0, jnp.minimum(s, 7), lane_pass, 0)
        return carry

    lax.fori_loop(1, 16, stage, 0)


def _sc_body(sidx_hbm, ssc_hbm, bx1_hbm, by1_hbm, bx2_hbm, by2_hbm,
             okx1, oky1, okx2, oky2, oks,
             idx_v, sc_v, gx1_v, gy1_v, gx2_v, gy2_v,
             kx1_v, ky1_v, kx2_v, ky2_v, kar_v, ks_v,
             sem):
    cid = lax.axis_index("c")
    sid = lax.axis_index("s")
    wid = cid * NS + sid

    @pl.when(wid < B)
    def _():
        b = wid
        pltpu.sync_copy(sidx_hbm.at[b], idx_v)
        pltpu.sync_copy(ssc_hbm.at[b], sc_v.at[pl.ds(0, TOPP)])
        cps = []
        for g in range(GCH):
            sl = pl.ds(g * C, C)
            row = idx_v.at[g]
            cps.append(pltpu.async_copy(bx1_hbm.at[row], gx1_v.at[sl], sem))
            cps.append(pltpu.async_copy(by1_hbm.at[row], gy1_v.at[sl], sem))
            cps.append(pltpu.async_copy(bx2_hbm.at[row], gx2_v.at[sl], sem))
            cps.append(pltpu.async_copy(by2_hbm.at[row], gy2_v.at[sl], sem))
        for cp in cps:
            cp.wait()
        zf = jnp.zeros((L,), jnp.float32)
        for i in range(KOUT // L):
            sl = pl.ds(i * L, L)
            kx1_v[sl] = zf
            ky1_v[sl] = zf
            kx2_v[sl] = zf
            ky2_v[sl] = zf
            kar_v[sl] = zf
            ks_v[sl] = zf
    lane = lax.iota(jnp.int32, L)
    lane0 = lane == 0
    t_max = jnp.where(wid < B, PRE_NMS_TOPN, 0)

    def nms_cond(st):
        t, n, jj = st
        return jnp.logical_and(t < t_max, n < POST_NMS_TOPN)

    def nms_body(st):
        t, n, jj = st
        tsl = pl.ds(t, L)
        vx1 = jnp.full((L,), gx1_v[tsl][0])
        vy1 = jnp.full((L,), gy1_v[tsl][0])
        vx2 = jnp.full((L,), gx2_v[tsl][0])
        vy2 = jnp.full((L,), gy2_v[tsl][0])
        varc = (vx2 - vx1 + 1.0) * (vy2 - vy1 + 1.0)
        sl = pl.ds(jj * L, L)
        xx1 = jnp.maximum(kx1_v[sl], vx1)
        yy1 = jnp.maximum(ky1_v[sl], vy1)
        xx2 = jnp.minimum(kx2_v[sl], vx2)
        yy2 = jnp.minimum(ky2_v[sl], vy2)
        ww = jnp.maximum(0.0, xx2 - xx1 + 1.0)
        hh = jnp.maximum(0.0, yy2 - yy1 + 1.0)
        inter = ww * hh
        iou = inter / ((kar_v[sl] + varc) - inter)
        m = (iou > NMS_THRESH) & ((jj * L + lane) < n)
        sup = jnp.any(m)
        done = jnp.logical_or(sup, (jj + 1) * L >= n)
        keep = jnp.logical_and(done, jnp.logical_not(sup))
        kmask = lane0 & keep
        nvec = jnp.full((L,), n, dtype=jnp.int32)
        plsc.store_scatter(kx1_v, [nvec], vx1, mask=kmask)
        plsc.store_scatter(ky1_v, [nvec], vy1, mask=kmask)
        plsc.store_scatter(kx2_v, [nvec], vx2, mask=kmask)
        plsc.store_scatter(ky2_v, [nvec], vy2, mask=kmask)
        plsc.store_scatter(kar_v, [nvec], varc, mask=kmask)
        plsc.store_scatter(ks_v, [nvec], sc_v[pl.ds(t, L)], mask=kmask)
        t2 = jnp.where(done, t + 1, t)
        n2 = n + keep.astype(jnp.int32)
        jj2 = jnp.where(done, 0, jj + 1)
        return t2, n2, jj2

    lax.while_loop(nms_cond, nms_body,
                   (jnp.int32(0), jnp.int32(0), jnp.int32(0)))

    @pl.when(wid < B)
    def _():
        b = wid
        pltpu.sync_copy(kx1_v, okx1.at[b])
        pltpu.sync_copy(ky1_v, oky1.at[b])
        pltpu.sync_copy(kx2_v, okx2.at[b])
        pltpu.sync_copy(ky2_v, oky2.at[b])
        pltpu.sync_copy(ks_v, oks.at[b])


def _prep(scores, bbox_deltas):
    sc = scores[:, A_NUM:, :, :].transpose(0, 2, 3, 1).reshape(B, N)
    dl = bbox_deltas.transpose(0, 2, 3, 1).reshape(B, N, 4)
    sc_p = jnp.concatenate(
        [sc, jnp.full((B, NP - N), -jnp.inf, jnp.float32)], axis=1)
    dl_p = jnp.concatenate(
        [dl, jnp.zeros((B, NP - N, 4), jnp.float32)], axis=1)
    arrs = {
        "sc": sc_p.reshape(B, R, C),
        "dx": dl_p[..., 0].reshape(B, R, C),
        "dy": dl_p[..., 1].reshape(B, R, C),
        "dw": dl_p[..., 2].reshape(B, R, C),
        "dh": dl_p[..., 3].reshape(B, R, C),
        "ax1": jnp.asarray(_ANCHORS[:, 0].reshape(R, C)),
        "ay1": jnp.asarray(_ANCHORS[:, 1].reshape(R, C)),
        "ax2": jnp.asarray(_ANCHORS[:, 2].reshape(R, C)),
        "ay2": jnp.asarray(_ANCHORS[:, 3].reshape(R, C)),
    }
    return arrs


def _tc_call(im_info, a, interpret=False):
    f32 = jnp.float32
    out_shape = [
        jax.ShapeDtypeStruct((B, R, C), f32),
        jax.ShapeDtypeStruct((B, R, C), jnp.int32),
        jax.ShapeDtypeStruct((B, R, C), f32),
        jax.ShapeDtypeStruct((B, R, C), f32),
        jax.ShapeDtypeStruct((B, R, C), f32),
        jax.ShapeDtypeStruct((B, R, C), f32),
    ]
    in_specs = [pl.BlockSpec(memory_space=pltpu.SMEM)] + \
        [pl.BlockSpec(memory_space=pltpu.VMEM)] * 9
    fn = pl.pallas_call(_tc_body, out_shape=out_shape, in_specs=in_specs,
                        interpret=interpret)
    return fn(im_info, a["sc"], a["dx"], a["dy"], a["dw"], a["dh"],
              a["ax1"], a["ay1"], a["ax2"], a["ay2"])


def kernel(scores, bbox_deltas, im_info):
    a = _prep(scores, bbox_deltas)
    key, idx, bx1, by1, bx2, by2 = _tc_call(im_info, a)

    sidx = idx.reshape(B, NP)[:, :TOPP].reshape(B, GCH, C)
    ssc = key.reshape(B, NP)[:, :TOPP]
    bx1f = bx1.reshape(-1)
    by1f = by1.reshape(-1)
    bx2f = bx2.reshape(-1)
    by2f = by2.reshape(-1)

    f32 = jnp.float32
    row = jax.ShapeDtypeStruct((B, KOUT), f32)
    nms = pl.kernel(
        _sc_body,
        out_type=[row, row, row, row, row],
        mesh=plsc.VectorSubcoreMesh(core_axis_name="c", subcore_axis_name="s"),
        compiler_params=pltpu.CompilerParams(needs_layout_passes=False),
        scratch_types=[pltpu.VMEM((GCH, C), jnp.int32)]
        + [pltpu.VMEM((TOPP + L,), f32)] * 5
        + [pltpu.VMEM((KOUT,), f32)] * 6
        + [pltpu.SemaphoreType.DMA],
    )
    kx1o, ky1o, kx2o, ky2o, kso = nms(sidx, ssc, bx1f, by1f, bx2f, by2f)

    kept = jnp.stack([kx1o, ky1o, kx2o, ky2o], axis=-1)[:, :POST_NMS_TOPN]
    bcol = jnp.broadcast_to(
        jnp.arange(B, dtype=f32)[:, None, None], (B, POST_NMS_TOPN, 1))
    output = jnp.concatenate([bcol, kept], axis=2)
    scores_single = kso[B - 1, :POST_NMS_TOPN].reshape(-1, 1)
    return output, scores_single


# speedup vs baseline: 1.1529x; 1.0649x over previous
import numpy as np
import jax
import jax.numpy as jnp
from jax import lax
from jax.experimental import pallas as pl
from jax.experimental.pallas import tpu as pltpu
from jax.experimental.pallas import tpu_sc as plsc

FEAT_STRIDE = 16
PRE_NMS_TOPN = 6000
POST_NMS_TOPN = 300
NMS_THRESH = 0.7
A_NUM = 9
B, H, W = 4, 50, 50
N = H * W * A_NUM
NP = 32768
C = 128
R = NP // C
TOPP = 6016
GCH = TOPP // C
KOUT = 512
L = 16
NC, NS = 2, 16


def _gen_all_anchors():
    base_size = 16.0
    ratios = np.array([0.5, 1.0, 2.0], dtype=np.float64)
    scales = np.array([8.0, 16.0, 32.0], dtype=np.float64)
    base = np.array([1, 1, base_size, base_size], dtype=np.float64) - 1
    w = base[2] - base[0] + 1
    h = base[3] - base[1] + 1
    x_ctr = base[0] + 0.5 * (w - 1)
    y_ctr = base[1] + 0.5 * (h - 1)
    size = w * h
    ws_r = np.round(np.sqrt(size / ratios))
    hs_r = np.round(ws_r * ratios)
    rows = []
    for i in range(3):
        ws = ws_r[i] * scales
        hs = hs_r[i] * scales
        for j in range(3):
            rows.append([x_ctr - 0.5 * (ws[j] - 1), y_ctr - 0.5 * (hs[j] - 1),
                         x_ctr + 0.5 * (ws[j] - 1), y_ctr + 0.5 * (hs[j] - 1)])
    anc = np.array(rows, dtype=np.float32)
    ys, xs = np.meshgrid(np.arange(H), np.arange(W), indexing="ij")
    sx = (xs * FEAT_STRIDE).ravel().astype(np.float32)
    sy = (ys * FEAT_STRIDE).ravel().astype(np.float32)
    shifts = np.stack([sx, sy, sx, sy], axis=1)
    full = (anc[None, :, :] + shifts[:, None, :]).reshape(N, 4)
    out = np.zeros((NP, 4), dtype=np.float32)
    out[:N] = full
    return out


_ANCHORS = _gen_all_anchors()


def _xor_rows(x, g):
    xr = x.reshape(B, R // (2 * g), 2, g, C)
    sw = jnp.concatenate([xr[:, :, 1:2], xr[:, :, 0:1]], axis=2)
    return sw.reshape(B, R, C)


def _xor_lanes(x, g, ci):
    up = pltpu.roll(x, C - g, 2)
    dn = pltpu.roll(x, g, 2)
    return jnp.where((ci & g) == 0, up, dn)


def _tc_body(iminfo_ref, sc_ref, dx_ref, dy_ref, dw_ref, dh_ref,
             ax1_ref, ay1_ref, ax2_ref, ay2_ref,
             key_ref, idx_ref, bx1_ref, by1_ref, bx2_ref, by2_ref):
    ax1 = ax1_ref[...]
    ay1 = ay1_ref[...]
    ax2 = ax2_ref[...]
    ay2 = ay2_ref[...]
    widths = ax2 - ax1 + 1.0
    heights = ay2 - ay1 + 1.0
    ctr_x = ax1 + 0.5 * widths
    ctr_y = ay1 + 0.5 * heights
    for b in range(B):
        pcx = dx_ref[b] * widths + ctr_x
        pcy = dy_ref[b] * heights + ctr_y
        pw = jnp.exp(dw_ref[b]) * widths
        ph = jnp.exp(dh_ref[b]) * heights
        x1 = pcx - 0.5 * pw
        y1 = pcy - 0.5 * ph
        x2 = pcx + 0.5 * pw
        y2 = pcy + 0.5 * ph
        hmax = iminfo_ref[b, 0] - 1.0
        wmax = iminfo_ref[b, 1] - 1.0
        bx1_ref[b] = jnp.clip(x1, 0.0, wmax)
        by1_ref[b] = jnp.clip(y1, 0.0, hmax)
        bx2_ref[b] = jnp.clip(x2, 0.0, wmax)
        by2_ref[b] = jnp.clip(y2, 0.0, hmax)

    ri = lax.broadcasted_iota(jnp.int32, (B, R, C), 1)
    ci = lax.broadcasted_iota(jnp.int32, (B, R, C), 2)
    bi = lax.broadcasted_iota(jnp.int32, (B, R, C), 0)
    pos = ri * C + ci
    key_ref[...] = sc_ref[...]
    idx_ref[...] = pos + bi * NP
    one = jnp.int32(1)

    def finish_pass(j, kv, pk, pi):
        key = key_ref[...]
        idx = idx_ref[...]
        lower = (pos & j) == 0
        region = (pos & kv) == 0
        self_first = (key > pk) | ((key == pk) & (idx < pi))
        take_self = (self_first == lower) == region
        key_ref[...] = jnp.where(take_self, key, pk)
        idx_ref[...] = jnp.where(take_self, idx, pi)

    def lane_pass_static(t, kv):
        j = 1 << t
        pk = _xor_lanes(key_ref[...], j, ci)
        pi = _xor_lanes(idx_ref[...], j, ci)
        finish_pass(j, kv, pk, pi)

    def row_pass_static(t, kv):
        j = 1 << t
        pk = _xor_rows(key_ref[...], j // C)
        pi = _xor_rows(idx_ref[...], j // C)
        finish_pass(j, kv, pk, pi)

    def stage(s, carry):
        kv = lax.shift_left(one, s)

        def row_pass(i, c2):
            tt = s - 1 - i
            lax.switch(tt - 7,
                       [(lambda t=t: row_pass_static(t, kv))
                        for t in range(7, 15)])
            return c2

        lax.fori_loop(0, jnp.maximum(s - 7, 0), row_pass, 0)

        def lane_pass(i, c2):
            tt = jnp.minimum(s - 1, 6) - i
            lax.switch(tt,
                       [(lambda t=t: lane_pass_static(t, kv))
                        for t in range(7)])
            return c2

        lax.fori_loop(0, jnp.minimum(s, 7), lane_pass, 0)
        return carry

    lax.fori_loop(1, 16, stage, 0)


def _sc_body(sidx_hbm, ssc_hbm, bx1_hbm, by1_hbm, bx2_hbm, by2_hbm,
             okx1, oky1, okx2, oky2, oks,
             idx_v, sc_v, gx1_v, gy1_v, gx2_v, gy2_v,
             kx1_v, ky1_v, kx2_v, ky2_v, kar_v, ks_v,
             sem):
    cid = lax.axis_index("c")
    sid = lax.axis_index("s")
    wid = cid * NS + sid

    @pl.when(wid < B)
    def _():
        b = wid
        pltpu.sync_copy(sidx_hbm.at[b], idx_v)
        pltpu.sync_copy(ssc_hbm.at[b], sc_v.at[pl.ds(0, TOPP)])
        cps = []
        for g in range(GCH):
            sl = pl.ds(g * C, C)
            row = idx_v.at[g]
            cps.append(pltpu.async_copy(bx1_hbm.at[row], gx1_v.at[sl], sem))
            cps.append(pltpu.async_copy(by1_hbm.at[row], gy1_v.at[sl], sem))
            cps.append(pltpu.async_copy(bx2_hbm.at[row], gx2_v.at[sl], sem))
            cps.append(pltpu.async_copy(by2_hbm.at[row], gy2_v.at[sl], sem))
        for cp in cps:
            cp.wait()
        zf = jnp.zeros((L,), jnp.float32)
        for i in range(KOUT // L):
            sl = pl.ds(i * L, L)
            kx1_v[sl] = zf
            ky1_v[sl] = zf
            kx2_v[sl] = zf
            ky2_v[sl] = zf
            kar_v[sl] = zf
            ks_v[sl] = zf
    lane = lax.iota(jnp.int32, L)
    lane0 = lane == 0
    t_max = jnp.where(wid < B, PRE_NMS_TOPN, 0)

    def nms_cond(st):
        t, n, jj = st
        return jnp.logical_and(t < t_max, n < POST_NMS_TOPN)

    def nms_body(st):
        t, n, jj = st
        tsl = pl.ds(t, L)
        vx1 = jnp.full((L,), gx1_v[tsl][0])
        vy1 = jnp.full((L,), gy1_v[tsl][0])
        vx2 = jnp.full((L,), gx2_v[tsl][0])
        vy2 = jnp.full((L,), gy2_v[tsl][0])
        varc = (vx2 - vx1 + 1.0) * (vy2 - vy1 + 1.0)
        sl = pl.ds(jj * L, L)
        xx1 = jnp.maximum(kx1_v[sl], vx1)
        yy1 = jnp.maximum(ky1_v[sl], vy1)
        xx2 = jnp.minimum(kx2_v[sl], vx2)
        yy2 = jnp.minimum(ky2_v[sl], vy2)
        ww = jnp.maximum(0.0, xx2 - xx1 + 1.0)
        hh = jnp.maximum(0.0, yy2 - yy1 + 1.0)
        inter = ww * hh
        iou = inter / ((kar_v[sl] + varc) - inter)
        m = (iou > NMS_THRESH) & ((jj * L + lane) < n)
        sup = jnp.any(m)
        done = jnp.logical_or(sup, (jj + 1) * L >= n)
        keep = jnp.logical_and(done, jnp.logical_not(sup))
        kmask = lane0 & keep
        nvec = jnp.full((L,), n, dtype=jnp.int32)
        plsc.store_scatter(kx1_v, [nvec], vx1, mask=kmask)
        plsc.store_scatter(ky1_v, [nvec], vy1, mask=kmask)
        plsc.store_scatter(kx2_v, [nvec], vx2, mask=kmask)
        plsc.store_scatter(ky2_v, [nvec], vy2, mask=kmask)
        plsc.store_scatter(kar_v, [nvec], varc, mask=kmask)
        plsc.store_scatter(ks_v, [nvec], sc_v[pl.ds(t, L)], mask=kmask)
        t2 = jnp.where(done, t + 1, t)
        n2 = n + keep.astype(jnp.int32)
        jj2 = jnp.where(done, 0, jj + 1)
        return t2, n2, jj2

    lax.while_loop(nms_cond, nms_body,
                   (jnp.int32(0), jnp.int32(0), jnp.int32(0)))

    @pl.when(wid < B)
    def _():
        b = wid
        pltpu.sync_copy(kx1_v, okx1.at[b])
        pltpu.sync_copy(ky1_v, oky1.at[b])
        pltpu.sync_copy(kx2_v, okx2.at[b])
        pltpu.sync_copy(ky2_v, oky2.at[b])
        pltpu.sync_copy(ks_v, oks.at[b])


def _prep(scores, bbox_deltas):
    sc = scores[:, A_NUM:, :, :].transpose(0, 2, 3, 1).reshape(B, N)
    dl = bbox_deltas.transpose(0, 2, 3, 1).reshape(B, N, 4)
    sc_p = jnp.concatenate(
        [sc, jnp.full((B, NP - N), -jnp.inf, jnp.float32)], axis=1)
    dl_p = jnp.concatenate(
        [dl, jnp.zeros((B, NP - N, 4), jnp.float32)], axis=1)
    arrs = {
        "sc": sc_p.reshape(B, R, C),
        "dx": dl_p[..., 0].reshape(B, R, C),
        "dy": dl_p[..., 1].reshape(B, R, C),
        "dw": dl_p[..., 2].reshape(B, R, C),
        "dh": dl_p[..., 3].reshape(B, R, C),
        "ax1": jnp.asarray(_ANCHORS[:, 0].reshape(R, C)),
        "ay1": jnp.asarray(_ANCHORS[:, 1].reshape(R, C)),
        "ax2": jnp.asarray(_ANCHORS[:, 2].reshape(R, C)),
        "ay2": jnp.asarray(_ANCHORS[:, 3].reshape(R, C)),
    }
    return arrs


def _tc_call(im_info, a, interpret=False):
    f32 = jnp.float32
    out_shape = [
        jax.ShapeDtypeStruct((B, R, C), f32),
        jax.ShapeDtypeStruct((B, R, C), jnp.int32),
        jax.ShapeDtypeStruct((B, R, C), f32),
        jax.ShapeDtypeStruct((B, R, C), f32),
        jax.ShapeDtypeStruct((B, R, C), f32),
        jax.ShapeDtypeStruct((B, R, C), f32),
    ]
    in_specs = [pl.BlockSpec(memory_space=pltpu.SMEM)] + \
        [pl.BlockSpec(memory_space=pltpu.VMEM)] * 9
    fn = pl.pallas_call(_tc_body, out_shape=out_shape, in_specs=in_specs,
                        interpret=interpret)
    return fn(im_info, a["sc"], a["dx"], a["dy"], a["dw"], a["dh"],
              a["ax1"], a["ay1"], a["ax2"], a["ay2"])


def kernel(scores, bbox_deltas, im_info):
    a = _prep(scores, bbox_deltas)
    key, idx, bx1, by1, bx2, by2 = _tc_call(im_info, a)

    sidx = idx.reshape(B, NP)[:, :TOPP].reshape(B, GCH, C)
    ssc = key.reshape(B, NP)[:, :TOPP]
    bx1f = bx1.reshape(-1)
    by1f = by1.reshape(-1)
    bx2f = bx2.reshape(-1)
    by2f = by2.reshape(-1)

    f32 = jnp.float32
    row = jax.ShapeDtypeStruct((B, KOUT), f32)
    nms = pl.kernel(
        _sc_body,
        out_type=[row, row, row, row, row],
        mesh=plsc.VectorSubcoreMesh(core_axis_name="c", subcore_axis_name="s"),
        compiler_params=pltpu.CompilerParams(needs_layout_passes=False),
        scratch_types=[pltpu.VMEM((GCH, C), jnp.int32)]
        + [pltpu.VMEM((TOPP + L,), f32)] * 5
        + [pltpu.VMEM((KOUT,), f32)] * 6
        + [pltpu.SemaphoreType.DMA],
    )
    kx1o, ky1o, kx2o, ky2o, kso = nms(sidx, ssc, bx1f, by1f, bx2f, by2f)

    kept = jnp.stack([kx1o, ky1o, kx2o, ky2o], axis=-1)[:, :POST_NMS_TOPN]
    bcol = jnp.broadcast_to(
        jnp.arange(B, dtype=f32)[:, None, None], (B, POST_NMS_TOPN, 1))
    output = jnp.concatenate([bcol, kept], axis=2)
    scores_single = kso[B - 1, :POST_NMS_TOPN].reshape(-1, 1)
    return output, scores_single
